# baseline (speedup 1.0000x reference)
import functools

import jax
import jax.numpy as jnp
from jax.experimental import pallas as pl
from jax.experimental.pallas import tpu as pltpu

_INV_SQRT2 = 0.7071067811865476
_MiB = 1024 * 1024


def _gelu(v):
    return 0.5 * v * (1.0 + jax.lax.erf(v * _INV_SQRT2))


def _se_tile_body(x_ref, w1_ref, w2_ref, o_ref, *, inv_s, s, mask_lanes):
    x = x_ref[...]
    if mask_lanes:
        lane = jax.lax.broadcasted_iota(jnp.int32, x.shape, 2)
        x = jnp.where(lane < s, x, 0.0)
    pooled = jnp.sum(x, axis=2) * inv_s

    h = jax.lax.dot_general(pooled, w1_ref[...], (((1,), (1,)), ((), ())),
                            preferred_element_type=jnp.float32)
    h = _gelu(h)
    z = jax.lax.dot_general(h, w2_ref[...], (((1,), (1,)), ((), ())),
                            preferred_element_type=jnp.float32)
    gate = jax.nn.sigmoid(z)

    o_ref[...] = x_ref[...] * gate[:, :, None]


def kernel(x, fc1_w, fc2_w):
    b, c, d, h, w = x.shape
    s = d * h * w
    cr = fc1_w.shape[0]
    x2 = x.reshape(b, c, s)

    s_pad = ((s + 127) // 128) * 128
    tile_bytes = c * s_pad * 4
    bt = 1
    grid = b // bt

    body = functools.partial(_se_tile_body, inv_s=1.0 / float(s), s=s,
                             mask_lanes=(s % 128 != 0))
    vmem_limit = min(56 * _MiB, 4 * bt * tile_bytes + 4 * c * cr * 4 + 2 * _MiB)

    out2 = pl.pallas_call(
        body,
        out_shape=jax.ShapeDtypeStruct((b, c, s), x.dtype),
        grid=(grid,),
        in_specs=[
            pl.BlockSpec((bt, c, s), lambda i: (i, 0, 0)),
            pl.BlockSpec((cr, c), lambda i: (0, 0)),
            pl.BlockSpec((c, cr), lambda i: (0, 0)),
        ],
        out_specs=pl.BlockSpec((bt, c, s), lambda i: (i, 0, 0)),
        compiler_params=pltpu.CompilerParams(
            dimension_semantics=("parallel",),
            vmem_limit_bytes=int(max(32 * _MiB, vmem_limit)),
        ),
        cost_estimate=pl.CostEstimate(
            flops=2 * b * c * s + 4 * b * c * cr,
            transcendentals=b * (c + cr),
            bytes_accessed=2 * b * c * s * 4 + 2 * c * cr * 4,
        ),
    )(x2, fc1_w, fc2_w)
    return out2.reshape(b, c, d, h, w)


# speedup vs baseline: 1.0076x; 1.0076x over previous
import functools

import jax
import jax.numpy as jnp
from jax.experimental import pallas as pl
from jax.experimental.pallas import tpu as pltpu

_INV_SQRT2 = 0.7071067811865476
_MiB = 1024 * 1024

_NBUF = 6
_LEAD = 3
_K = 2


def _gelu(v):
    return 0.5 * v * (1.0 + jax.lax.erf(v * _INV_SQRT2))


def _excite(x, w1, w2, inv_s):
    pooled = jnp.sum(x, axis=2) * inv_s
    h = jax.lax.dot_general(pooled, w1, (((1,), (1,)), ((), ())),
                            preferred_element_type=jnp.float32)
    z = jax.lax.dot_general(_gelu(h), w2, (((1,), (1,)), ((), ())),
                            preferred_element_type=jnp.float32)
    return jax.nn.sigmoid(z)


def _se_stream_kernel(x_hbm, w1_ref, w2_ref, o_hbm, buf, rsem, wsem,
                      *, nb, c, inv_s):
    base = pl.program_id(0) * nb
    ck = c // _K

    def _chunk_copy(i, slot, k, sem, to_hbm):
        hslab = x_hbm if not to_hbm else o_hbm
        h_ref = hslab.at[pl.ds(base + i, 1), pl.ds(k * ck, ck), :]
        v_ref = buf.at[pl.ds(slot, 1), pl.ds(k * ck, ck), :]
        if to_hbm:
            return pltpu.make_async_copy(v_ref, h_ref, sem)
        return pltpu.make_async_copy(h_ref, v_ref, sem)

    def _start_read(i, slot):
        for k in range(_K):
            _chunk_copy(i, slot, k, rsem.at[slot, k], False).start()

    for j in range(min(_LEAD, nb)):
        _start_read(j, j % _NBUF)

    def _step(i, carry):
        slot = jax.lax.rem(i, _NBUF)
        for k in range(_K):
            _chunk_copy(i, slot, k, rsem.at[slot, k], False).wait()

        tile = buf.at[pl.ds(slot, 1)]
        x = tile[...]
        gate = _excite(x, w1_ref[...], w2_ref[...], inv_s)
        tile[...] = x * gate[:, :, None]

        for k in range(_K):
            _chunk_copy(i, slot, k, wsem.at[slot, k], True).start()

        nxt = i + _LEAD

        @pl.when(nxt < nb)
        def _():
            nslot = jax.lax.rem(nxt, _NBUF)

            @pl.when(nxt >= _NBUF)
            def _():
                for k in range(_K):
                    _chunk_copy(nxt - _NBUF, nslot, k,
                                wsem.at[nslot, k], True).wait()

            _start_read(nxt, nslot)

        return carry

    jax.lax.fori_loop(0, nb, _step, 0)

    for j in range(max(0, nb - _NBUF), nb):
        slot = j % _NBUF
        for k in range(_K):
            _chunk_copy(j, slot, k, wsem.at[slot, k], True).wait()


def kernel(x, fc1_w, fc2_w):
    b, c, d, h, w = x.shape
    s = d * h * w
    cr = fc1_w.shape[0]
    x2 = x.reshape(b, c, s)

    ncores = 2 if b % 2 == 0 else 1
    nb = b // ncores

    body = functools.partial(_se_stream_kernel, nb=nb, c=c,
                             inv_s=1.0 / float(s))

    out2 = pl.pallas_call(
        body,
        out_shape=jax.ShapeDtypeStruct((b, c, s), x.dtype),
        grid=(ncores,),
        in_specs=[
            pl.BlockSpec(memory_space=pl.ANY),
            pl.BlockSpec((cr, c), lambda i: (0, 0)),
            pl.BlockSpec((c, cr), lambda i: (0, 0)),
        ],
        out_specs=pl.BlockSpec(memory_space=pl.ANY),
        scratch_shapes=[
            pltpu.VMEM((_NBUF, c, s), jnp.float32),
            pltpu.SemaphoreType.DMA((_NBUF, _K)),
            pltpu.SemaphoreType.DMA((_NBUF, _K)),
        ],
        compiler_params=pltpu.CompilerParams(
            dimension_semantics=("parallel",),
            vmem_limit_bytes=int(_NBUF * c * s * 4 + 4 * c * cr * 4 + 4 * _MiB),
        ),
        cost_estimate=pl.CostEstimate(
            flops=2 * b * c * s + 4 * b * c * cr,
            transcendentals=b * (c + cr),
            bytes_accessed=2 * b * c * s * 4 + 2 * c * cr * 4,
        ),
    )(x2, fc1_w, fc2_w)
    return out2.reshape(b, c, d, h, w)
